# Initial kernel scaffold; baseline (speedup 1.0000x reference)
import jax
import jax.numpy as jnp
from jax.experimental import pallas as pl


def kernel(inputs, idx_inputs, cat_mask, numeric_mask):
    raise NotImplementedError("write your pallas kernel here")


# speedup vs baseline: 1.8587x; 1.8587x over previous
import functools

import jax
import jax.numpy as jnp
from jax import lax
from jax.experimental import pallas as pl
from jax.experimental.pallas import tpu as pltpu
from jax.experimental.pallas import tpu_sc as plsc

N = 262144
NUM_CAT = 26
NUM_ATTR = 154
NUM_NUM = 128
K = 32
NUM_GROUPS = 65536

TC_B = 1024

NS = 16
ROWS_PER_TILE = N // NS
CHUNK = 1024
SUB = 128
N_CHUNKS = ROWS_PER_TILE // CHUNK
SUBS_PER_CHUNK = CHUNK // SUB
KH = K // 2


def _tc_body(x_ref, idx_ref, p_ref, m_ref, v0_ref, v1_ref, g_ref):
    x = x_ref[...]
    p = p_ref[...]
    v = lax.dot(x, p, precision=lax.Precision.HIGHEST,
                preferred_element_type=jnp.float32)
    v0_ref[...] = v[:, :KH]
    v1_ref[...] = v[:, KH:]
    idx = idx_ref[...]
    m = m_ref[...]
    g_ref[...] = jnp.sum(idx * m, axis=1, keepdims=True)


def _tc_select(inputs, idx_inputs, p_full, mask26):
    grid = (N // TC_B,)
    return pl.pallas_call(
        _tc_body,
        grid=grid,
        in_specs=[
            pl.BlockSpec((TC_B, NUM_ATTR), lambda i: (i, 0)),
            pl.BlockSpec((TC_B, NUM_CAT), lambda i: (i, 0)),
            pl.BlockSpec((NUM_ATTR, K), lambda i: (0, 0)),
            pl.BlockSpec((1, NUM_CAT), lambda i: (0, 0)),
        ],
        out_specs=[
            pl.BlockSpec((TC_B, KH), lambda i: (i, 0)),
            pl.BlockSpec((TC_B, KH), lambda i: (i, 0)),
            pl.BlockSpec((TC_B, 1), lambda i: (i, 0)),
        ],
        out_shape=[
            jax.ShapeDtypeStruct((N, KH), jnp.float32),
            jax.ShapeDtypeStruct((N, KH), jnp.float32),
            jax.ShapeDtypeStruct((N, 1), jnp.int32),
        ],
        compiler_params=pltpu.CompilerParams(
            dimension_semantics=("parallel",)),
    )(inputs, idx_inputs, p_full, mask26)


def _sc_aggregate(vals0, vals1, gidx2d):
    mesh = plsc.VectorSubcoreMesh(core_axis_name="c", subcore_axis_name="s")

    @functools.partial(
        pl.kernel,
        mesh=mesh,
        out_type=[
            jax.ShapeDtypeStruct((N, KH), jnp.float32),
            jax.ShapeDtypeStruct((N, KH), jnp.float32),
        ],
        scratch_types=[
            pltpu.VMEM_SHARED((NUM_GROUPS, KH), jnp.float32),
            pltpu.VMEM((SUB, SUB), jnp.int32),
            pltpu.VMEM((CHUNK, KH), jnp.float32),
            pltpu.VMEM((CHUNK, KH), jnp.float32),
        ],
        compiler_params=pltpu.CompilerParams(use_tc_tiling_on_sc=False),
    )
    def k(v0_hbm, v1_hbm, g_hbm, o0_hbm, o1_hbm, agg, gidx_v, vals_v, out_v):
        c = lax.axis_index("c")
        s = lax.axis_index("s")

        def zrow(i, _):
            vals_v[i, :] = jnp.zeros((16,), jnp.float32)
            return 0
        lax.fori_loop(0, CHUNK, zrow, 0)
        zslice = NUM_GROUPS // NS
        for z in range(zslice // CHUNK):
            pltpu.sync_copy(vals_v,
                            agg.at[pl.ds(s * zslice + z * CHUNK, CHUNK)])

        pltpu.sync_copy(g_hbm.at[pl.ds(s * SUB, SUB)], gidx_v)
        plsc.subcore_barrier()

        def p1(chunk, _):
            rr = s * ROWS_PER_TILE + chunk * CHUNK

            @pl.when(c == 0)
            def _():
                pltpu.sync_copy(v0_hbm.at[pl.ds(rr, CHUNK)], vals_v)

            @pl.when(c == 1)
            def _():
                pltpu.sync_copy(v1_hbm.at[pl.ds(rr, CHUNK)], vals_v)

            for j in range(SUBS_PER_CHUNK):
                idxrow = gidx_v.at[chunk * SUBS_PER_CHUNK + j]
                pltpu.sync_copy(vals_v.at[pl.ds(j * SUB, SUB)],
                                agg.at[idxrow], add=True)
            return 0
        lax.fori_loop(0, N_CHUNKS, p1, 0)
        plsc.subcore_barrier()

        def p2(chunk, _):
            rr = s * ROWS_PER_TILE + chunk * CHUNK
            for j in range(SUBS_PER_CHUNK):
                idxrow = gidx_v.at[chunk * SUBS_PER_CHUNK + j]
                pltpu.sync_copy(agg.at[idxrow],
                                out_v.at[pl.ds(j * SUB, SUB)])

            @pl.when(c == 0)
            def _():
                pltpu.sync_copy(out_v, o0_hbm.at[pl.ds(rr, CHUNK)])

            @pl.when(c == 1)
            def _():
                pltpu.sync_copy(out_v, o1_hbm.at[pl.ds(rr, CHUNK)])
            return 0
        lax.fori_loop(0, N_CHUNKS, p2, 0)

    return k(vals0, vals1, gidx2d)


def kernel(inputs, idx_inputs, cat_mask, numeric_mask):
    cm = jax.nn.softmax(cat_mask)
    top_cat_val, top_idx = lax.top_k(cm, 1)
    nm = jax.nn.softmax(numeric_mask)
    top_num_val, top_nidx = lax.top_k(nm, K)
    scale = (top_num_val + top_cat_val) * 0.5
    p_full = jnp.zeros((NUM_ATTR, K), jnp.float32).at[
        NUM_CAT + top_nidx, jnp.arange(K)].set(scale)
    mask26 = (jnp.arange(NUM_CAT)[None, :] == top_idx[0]).astype(jnp.int32)

    vals0, vals1, gidx = _tc_select(
        inputs, idx_inputs.astype(jnp.int32), p_full, mask26)
    gidx2d = gidx.reshape(N // SUB, SUB)
    out0, out1 = _sc_aggregate(vals0, vals1, gidx2d)
    return jnp.concatenate([out0, out1], axis=1)
